# baseline (speedup 1.0000x reference)
import functools

import jax
import jax.numpy as jnp
from jax import lax
from jax.experimental import pallas as pl
from jax.experimental.pallas import tpu as pltpu
from jax.experimental.pallas import tpu_sc as plsc

_EPS = 1e-12
_LANES = 16

_GATHER_DNUMS = lax.GatherDimensionNumbers(
    offset_dims=(), collapsed_slice_dims=(0,), start_index_map=(0,))


def _shuffle(x, k):
    perm = lax.iota(jnp.int32, _LANES) ^ k
    return lax.gather(x, perm[:, None], _GATHER_DNUMS, (1,),
                      mode=lax.GatherScatterMode.PROMISE_IN_BOUNDS)


def _allsum(x):
    for k in (8, 4, 2, 1):
        x = x + _shuffle(x, k)
    return x


def _perm(x, perm):
    return lax.gather(x, perm[:, None], _GATHER_DNUMS, (1,),
                      mode=lax.GatherScatterMode.PROMISE_IN_BOUNDS)


def _stat_sums(acc, acc2):
    iota = lax.iota(jnp.int32, _LANES)
    a8 = acc + _shuffle(acc, 8)
    b8 = acc2 + _shuffle(acc2, 8)
    packed = jnp.where(iota < 8, a8, b8)
    for k in (4, 2, 1):
        packed = packed + _shuffle(packed, k)
    tot = _perm(packed, iota & 7)
    tot2 = _perm(packed, iota | 8)
    return tot, tot2


def _treesum(xs):
    while len(xs) > 1:
        xs = [a + b for a, b in zip(xs[::2], xs[1::2])]
    return xs[0]


def _rsqrt(v16):
    i = lax.bitcast_convert_type(v16, jnp.int32)
    y = lax.bitcast_convert_type(jnp.int32(0x5F375A86) - (i >> 1), jnp.float32)
    return y * (1.5 - 0.5 * v16 * y * y)


def _make_sc_kernel(N, S, H, V, C, n_workers):
    tpw = N // n_workers
    n_chunks = tpw // C
    n_half = n_chunks // 2
    n_grp = C // _LANES
    hs = H // _LANES

    mesh = plsc.VectorSubcoreMesh(core_axis_name="c", subcore_axis_name="s")

    @functools.partial(
        pl.kernel,
        out_type=jax.ShapeDtypeStruct((N, H), jnp.float32),
        mesh=mesh,
        scratch_types=[
            pltpu.VMEM((tpw,), jnp.int32),
            pltpu.VMEM((tpw,), jnp.int32),
            pltpu.VMEM((C, H), jnp.float32), pltpu.VMEM((C, H), jnp.float32),
            pltpu.VMEM((C, H), jnp.float32), pltpu.VMEM((C, H), jnp.float32),
            pltpu.VMEM((S, H), jnp.float32),
            pltpu.VMEM((2, H), jnp.float32),
            pltpu.VMEM((H,), jnp.float32),
            pltpu.VMEM((H,), jnp.float32),
            pltpu.SemaphoreType.DMA, pltpu.SemaphoreType.DMA,
            pltpu.SemaphoreType.DMA, pltpu.SemaphoreType.DMA,
        ],
    )
    def sc_kernel(ids_hbm, tts_hbm, w_hbm, p_hbm, t_hbm, g_hbm, b_hbm,
                  out_hbm, idx_v, tt_v, rows0, rows1, out0, out1,
                  pos_v, ttab_v, g_v, b_v, sem_g0, sem_g1, sem_w0, sem_w1):
        info = plsc.get_sparse_core_info()
        wid = lax.axis_index("s") * info.num_cores + lax.axis_index("c")
        base_w = wid * tpw

        K = 4
        CK = C // K

        class _Multi:
            def __init__(self, descs):
                self.descs = descs

            def start(self):
                for d in self.descs:
                    d.start()

            def wait(self):
                for d in self.descs:
                    d.wait()

        def gather(ci, rows, sem):
            return _Multi([
                pltpu.make_async_copy(
                    w_hbm.at[idx_v.at[pl.ds(ci * C + m * CK, CK)]],
                    rows.at[pl.ds(m * CK, CK)], sem)
                for m in range(K)])

        def writeback(ci, out, sem):
            return _Multi([
                pltpu.make_async_copy(
                    out.at[pl.ds(m * (C // 2), C // 2)],
                    out_hbm.at[pl.ds(base_w + ci * C + m * (C // 2), C // 2)],
                    sem)
                for m in range(2)])

        pltpu.sync_copy(ids_hbm.at[pl.ds(base_w, tpw)], idx_v)
        gather(0, rows0, sem_g0).start()
        pltpu.sync_copy(tts_hbm.at[pl.ds(base_w, tpw)], tt_v)
        pltpu.sync_copy(p_hbm.at[pl.ds(0, S)], pos_v)
        pltpu.sync_copy(t_hbm, ttab_v)
        pltpu.sync_copy(g_hbm, g_v)
        pltpu.sync_copy(b_hbm, b_v)

        def fold_body(s, carry):
            for h in range(hs):
                sl = pl.ds(h * _LANES, _LANES)
                pos_v[s, sl] = pos_v[s, sl] + ttab_v[0, sl]
            return carry
        lax.fori_loop(0, S, fold_body, 0)

        d8, g8, b8 = [], [], []
        for h in range(hs):
            sl = pl.ds(h * _LANES, _LANES)
            d8.append(ttab_v[1, sl] - ttab_v[0, sl])
            g8.append(g_v[sl])
            b8.append(b_v[sl])

        def compute(ci, rows, out):
            def grp_body(gi, carry):
                tg = gi * _LANES
                mf16 = tt_v[pl.ds(ci * C + tg, _LANES)].astype(jnp.float32)
                sg = lax.rem(ci * C + tg, S)
                for j in range(_LANES):
                    t = tg + j
                    s = lax.rem(sg + j, S)
                    mf = mf16[j]
                    xs = []
                    for h in range(hs):
                        sl = pl.ds(h * _LANES, _LANES)
                        x = rows[t, sl] + pos_v[s, sl] + mf * d8[h]
                        xs.append(x)
                    acc = _treesum(xs)
                    acc2 = _treesum([x * x for x in xs])
                    mean = _allsum(acc) * (1.0 / H)
                    var = _allsum(acc2) * (1.0 / H) - mean * mean
                    rstd = _rsqrt(var + _EPS)
                    for h in range(hs):
                        sl = pl.ds(h * _LANES, _LANES)
                        out[t, sl] = (xs[h] - mean) * rstd * g8[h] + b8[h]
                return carry
            lax.fori_loop(0, n_grp, grp_body, 0)

        gather(0, rows0, sem_g0).wait()

        def pipe_body(it, carry):
            ci0 = it * 2
            ci1 = ci0 + 1

            gather(ci1, rows1, sem_g1).start()

            @pl.when(it > 0)
            def _():
                writeback(ci0, out0, sem_w0).wait()

            compute(ci0, rows0, out0)
            writeback(ci0, out0, sem_w0).start()
            gather(ci1, rows1, sem_g1).wait()

            @pl.when(it < n_half - 1)
            def _():
                gather(ci0 + 2, rows0, sem_g0).start()

            @pl.when(it > 0)
            def _():
                writeback(ci1, out1, sem_w1).wait()

            compute(ci1, rows1, out1)
            writeback(ci1, out1, sem_w1).start()

            @pl.when(it < n_half - 1)
            def _():
                gather(ci0 + 2, rows0, sem_g0).wait()

            return carry

        lax.fori_loop(0, n_half, pipe_body, 0)

        writeback(n_chunks - 2, out0, sem_w0).wait()
        writeback(n_chunks - 1, out1, sem_w1).wait()

    return sc_kernel


def kernel(input_ids, token_type_ids, word_embeddings, position_embeddings,
           token_type_embeddings, gamma, beta):
    B, S = input_ids.shape
    V, H = word_embeddings.shape
    N = B * S
    info = plsc.get_sparse_core_info()
    n_workers = info.num_cores * info.num_subcores
    C = 160
    sc = _make_sc_kernel(N, S, H, V, C, n_workers)
    out = sc(
        input_ids.reshape(N),
        token_type_ids.reshape(N),
        word_embeddings,
        position_embeddings,
        token_type_embeddings,
        gamma,
        beta,
    )
    return out.reshape(B, S, H)


# speedup vs baseline: 1.1817x; 1.1817x over previous
import functools

import jax
import jax.numpy as jnp
from jax import lax
from jax.experimental import pallas as pl
from jax.experimental.pallas import tpu as pltpu
from jax.experimental.pallas import tpu_sc as plsc

_EPS = 1e-12
_LANES = 16

_GATHER_DNUMS = lax.GatherDimensionNumbers(
    offset_dims=(), collapsed_slice_dims=(0,), start_index_map=(0,))


def _shuffle(x, k):
    perm = lax.iota(jnp.int32, _LANES) ^ k
    return lax.gather(x, perm[:, None], _GATHER_DNUMS, (1,),
                      mode=lax.GatherScatterMode.PROMISE_IN_BOUNDS)


def _allsum(x):
    for k in (8, 4, 2, 1):
        x = x + _shuffle(x, k)
    return x


def _treesum(xs):
    while len(xs) > 1:
        xs = [a + b for a, b in zip(xs[::2], xs[1::2])]
    return xs[0]


def _rsqrt(v16):
    i = lax.bitcast_convert_type(v16, jnp.int32)
    y = lax.bitcast_convert_type(jnp.int32(0x5F375A86) - (i >> 1), jnp.float32)
    return y * (1.5 - 0.5 * v16 * y * y)


def _make_sc_kernel(N, S, H, V, C, n_workers):
    tpw = N // n_workers
    n_chunks = tpw // C
    n_half = n_chunks // 2
    n_grp = C // _LANES
    n_tail = C - n_grp * _LANES
    hs = H // _LANES

    mesh = plsc.VectorSubcoreMesh(core_axis_name="c", subcore_axis_name="s")

    @functools.partial(
        pl.kernel,
        out_type=jax.ShapeDtypeStruct((N, H), jnp.float32),
        mesh=mesh,
        scratch_types=[
            pltpu.VMEM((C,), jnp.int32), pltpu.VMEM((C,), jnp.int32),
            pltpu.VMEM((C,), jnp.int32), pltpu.VMEM((C,), jnp.int32),
            pltpu.VMEM((C, H), jnp.float32), pltpu.VMEM((C, H), jnp.float32),
            pltpu.VMEM((C, H), jnp.float32),
            pltpu.VMEM((S, H), jnp.float32),
            pltpu.VMEM((2, H), jnp.float32),
            pltpu.VMEM((H,), jnp.float32),
            pltpu.VMEM((H,), jnp.float32),
            pltpu.SemaphoreType.DMA, pltpu.SemaphoreType.DMA,
            pltpu.SemaphoreType.DMA, pltpu.SemaphoreType.DMA,
        ],
    )
    def sc_kernel(ids_hbm, tts_hbm, w_hbm, p_hbm, t_hbm, g_hbm, b_hbm,
                  out_hbm, idxb0, idxb1, ttb0, ttb1, rows0, rows1, outb,
                  pos_v, ttab_v, g_v, b_v, sem_g0, sem_g1, sem_w0, sem_w1):
        info = plsc.get_sparse_core_info()
        wid = lax.axis_index("s") * info.num_cores + lax.axis_index("c")
        base_w = wid * tpw

        def copy_ids(ci, idx, tt):
            pltpu.sync_copy(ids_hbm.at[pl.ds(base_w + ci * C, C)], idx)
            pltpu.sync_copy(tts_hbm.at[pl.ds(base_w + ci * C, C)], tt)

        def gather(idx, rows, sem):
            return pltpu.make_async_copy(w_hbm.at[idx], rows, sem)

        def writeback(ci, out, sem):
            return pltpu.make_async_copy(
                out, out_hbm.at[pl.ds(base_w + ci * C, C)], sem)

        copy_ids(0, idxb0, ttb0)
        gather(idxb0, rows0, sem_g0).start()
        pltpu.sync_copy(p_hbm.at[pl.ds(0, S)], pos_v)
        pltpu.sync_copy(t_hbm, ttab_v)
        pltpu.sync_copy(g_hbm, g_v)
        pltpu.sync_copy(b_hbm, b_v)

        def fold_body(s, carry):
            for h in range(hs):
                sl = pl.ds(h * _LANES, _LANES)
                pos_v[s, sl] = pos_v[s, sl] + ttab_v[0, sl]
            return carry
        lax.fori_loop(0, S, fold_body, 0)

        d8, g8, b8 = [], [], []
        for h in range(hs):
            sl = pl.ds(h * _LANES, _LANES)
            d8.append(ttab_v[1, sl] - ttab_v[0, sl])
            g8.append(g_v[sl])
            b8.append(b_v[sl])

        def compute(rows, tt, out):
            def token(t, mf):
                xs = []
                for h in range(hs):
                    sl = pl.ds(h * _LANES, _LANES)
                    x = rows[t, sl] + pos_v[t, sl] + mf * d8[h]
                    xs.append(x)
                acc = _treesum(xs)
                acc2 = _treesum([x * x for x in xs])
                mean = _allsum(acc) * (1.0 / H)
                var = _allsum(acc2) * (1.0 / H) - mean * mean
                rstd = _rsqrt(var + _EPS)
                for h in range(hs):
                    sl = pl.ds(h * _LANES, _LANES)
                    out[t, sl] = (xs[h] - mean) * rstd * g8[h] + b8[h]

            def grp_body(gi, carry):
                tg = gi * _LANES
                mf16 = tt[pl.ds(tg, _LANES)].astype(jnp.float32)
                for j in range(_LANES):
                    token(tg + j, mf16[j])
                return carry
            lax.fori_loop(0, n_grp, grp_body, 0)

            if n_tail:
                mf16 = tt[pl.ds(C - _LANES, _LANES)].astype(jnp.float32)
                for j in range(n_tail):
                    token(n_grp * _LANES + j, mf16[j + _LANES - n_tail])

        gather(idxb0, rows0, sem_g0).wait()

        def pipe_body(it, carry):
            ci0 = it * 2
            ci1 = ci0 + 1

            copy_ids(ci1, idxb1, ttb1)
            gather(idxb1, rows1, sem_g1).start()

            compute(rows0, ttb0, outb)
            writeback(ci0, outb, sem_w0).start()
            gather(idxb1, rows1, sem_g1).wait()

            @pl.when(it < n_half - 1)
            def _():
                copy_ids(ci0 + 2, idxb0, ttb0)
                gather(idxb0, rows0, sem_g0).start()

            writeback(ci0, outb, sem_w0).wait()
            compute(rows1, ttb1, outb)
            writeback(ci1, outb, sem_w1).start()

            @pl.when(it < n_half - 1)
            def _():
                gather(idxb0, rows0, sem_g0).wait()

            writeback(ci1, outb, sem_w1).wait()
            return carry

        lax.fori_loop(0, n_half, pipe_body, 0)

    return sc_kernel


def kernel(input_ids, token_type_ids, word_embeddings, position_embeddings,
           token_type_embeddings, gamma, beta):
    B, S = input_ids.shape
    V, H = word_embeddings.shape
    N = B * S
    info = plsc.get_sparse_core_info()
    n_workers = info.num_cores * info.num_subcores
    C = S
    sc = _make_sc_kernel(N, S, H, V, C, n_workers)
    out = sc(
        input_ids.reshape(N),
        token_type_ids.reshape(N),
        word_embeddings,
        position_embeddings,
        token_type_embeddings,
        gamma,
        beta,
    )
    return out.reshape(B, S, H)


# speedup vs baseline: 1.1963x; 1.0123x over previous
import functools

import jax
import jax.numpy as jnp
from jax import lax
from jax.experimental import pallas as pl
from jax.experimental.pallas import tpu as pltpu
from jax.experimental.pallas import tpu_sc as plsc

_EPS = 1e-12
_LANES = 16

_GATHER_DNUMS = lax.GatherDimensionNumbers(
    offset_dims=(), collapsed_slice_dims=(0,), start_index_map=(0,))


def _shuffle(x, k):
    perm = lax.iota(jnp.int32, _LANES) ^ k
    return lax.gather(x, perm[:, None], _GATHER_DNUMS, (1,),
                      mode=lax.GatherScatterMode.PROMISE_IN_BOUNDS)


def _allsum(x):
    for k in (8, 4, 2, 1):
        x = x + _shuffle(x, k)
    return x


def _treesum(xs):
    while len(xs) > 1:
        xs = [a + b for a, b in zip(xs[::2], xs[1::2])]
    return xs[0]


def _rsqrt(v16):
    i = lax.bitcast_convert_type(v16, jnp.int32)
    y = lax.bitcast_convert_type(jnp.int32(0x5F375A86) - (i >> 1), jnp.float32)
    return y * (1.5 - 0.5 * v16 * y * y)


def _make_sc_kernel(N, S, H, V, C, n_workers):
    tpw = N // n_workers
    n_chunks = tpw // C
    n_half = n_chunks // 2
    n_grp = C // _LANES
    n_tail = C - n_grp * _LANES
    hs = H // _LANES

    mesh = plsc.VectorSubcoreMesh(core_axis_name="c", subcore_axis_name="s")

    @functools.partial(
        pl.kernel,
        out_type=jax.ShapeDtypeStruct((N, H), jnp.float32),
        mesh=mesh,
        scratch_types=[
            pltpu.VMEM((C,), jnp.int32), pltpu.VMEM((C,), jnp.int32),
            pltpu.VMEM((C,), jnp.int32), pltpu.VMEM((C,), jnp.int32),
            pltpu.VMEM((C, H), jnp.float32), pltpu.VMEM((C, H), jnp.float32),
            pltpu.VMEM((C, H), jnp.float32),
            pltpu.VMEM((S, H), jnp.float32),
            pltpu.VMEM((2, H), jnp.float32),
            pltpu.VMEM((H,), jnp.float32),
            pltpu.VMEM((H,), jnp.float32),
            pltpu.SemaphoreType.DMA, pltpu.SemaphoreType.DMA,
            pltpu.SemaphoreType.DMA, pltpu.SemaphoreType.DMA,
        ],
    )
    def sc_kernel(ids_hbm, tts_hbm, w_hbm, p_hbm, t_hbm, g_hbm, b_hbm,
                  out_hbm, idxb0, idxb1, ttb0, ttb1, rows0, rows1, outb,
                  pos_v, ttab_v, g_v, b_v, sem_g0, sem_g1, sem_w0, sem_w1):
        info = plsc.get_sparse_core_info()
        wid = lax.axis_index("s") * info.num_cores + lax.axis_index("c")
        base_w = wid * tpw

        def copy_ids(ci, idx, tt):
            pltpu.sync_copy(ids_hbm.at[pl.ds(base_w + ci * C, C)], idx)
            pltpu.sync_copy(tts_hbm.at[pl.ds(base_w + ci * C, C)], tt)

        K = 5
        CK = C // K

        class _Multi:
            def __init__(self, descs):
                self.descs = descs

            def start(self):
                for d in self.descs:
                    d.start()

            def wait(self):
                for d in self.descs:
                    d.wait()

        def gather(idx, rows, sem):
            return _Multi([
                pltpu.make_async_copy(
                    w_hbm.at[idx.at[pl.ds(m * CK, CK)]],
                    rows.at[pl.ds(m * CK, CK)], sem)
                for m in range(K)])

        def writeback(ci, out, sem):
            return pltpu.make_async_copy(
                out, out_hbm.at[pl.ds(base_w + ci * C, C)], sem)

        copy_ids(0, idxb0, ttb0)
        gather(idxb0, rows0, sem_g0).start()
        pltpu.sync_copy(p_hbm.at[pl.ds(0, S)], pos_v)
        pltpu.sync_copy(t_hbm, ttab_v)
        pltpu.sync_copy(g_hbm, g_v)
        pltpu.sync_copy(b_hbm, b_v)

        def fold_body(s, carry):
            for h in range(hs):
                sl = pl.ds(h * _LANES, _LANES)
                pos_v[s, sl] = pos_v[s, sl] + ttab_v[0, sl]
            return carry
        lax.fori_loop(0, S, fold_body, 0)

        d8, g8, b8 = [], [], []
        for h in range(hs):
            sl = pl.ds(h * _LANES, _LANES)
            d8.append(ttab_v[1, sl] - ttab_v[0, sl])
            g8.append(g_v[sl])
            b8.append(b_v[sl])

        def compute(rows, tt, out):
            def token(t, mf):
                xs = []
                for h in range(hs):
                    sl = pl.ds(h * _LANES, _LANES)
                    x = rows[t, sl] + pos_v[t, sl] + mf * d8[h]
                    xs.append(x)
                acc = _treesum(xs)
                acc2 = _treesum([x * x for x in xs])
                mean = _allsum(acc) * (1.0 / H)
                var = _allsum(acc2) * (1.0 / H) - mean * mean
                rstd = _rsqrt(var + _EPS)
                for h in range(hs):
                    sl = pl.ds(h * _LANES, _LANES)
                    out[t, sl] = (xs[h] - mean) * rstd * g8[h] + b8[h]

            def grp_body(gi, carry):
                tg = gi * _LANES
                mf16 = tt[pl.ds(tg, _LANES)].astype(jnp.float32)
                for j in range(_LANES):
                    token(tg + j, mf16[j])
                return carry
            lax.fori_loop(0, n_grp, grp_body, 0)

            if n_tail:
                mf16 = tt[pl.ds(C - _LANES, _LANES)].astype(jnp.float32)
                for j in range(n_tail):
                    token(n_grp * _LANES + j, mf16[j + _LANES - n_tail])

        gather(idxb0, rows0, sem_g0).wait()

        def pipe_body(it, carry):
            ci0 = it * 2
            ci1 = ci0 + 1

            copy_ids(ci1, idxb1, ttb1)
            gather(idxb1, rows1, sem_g1).start()

            compute(rows0, ttb0, outb)
            writeback(ci0, outb, sem_w0).start()
            gather(idxb1, rows1, sem_g1).wait()

            @pl.when(it < n_half - 1)
            def _():
                copy_ids(ci0 + 2, idxb0, ttb0)
                gather(idxb0, rows0, sem_g0).start()

            writeback(ci0, outb, sem_w0).wait()
            compute(rows1, ttb1, outb)
            writeback(ci1, outb, sem_w1).start()

            @pl.when(it < n_half - 1)
            def _():
                gather(idxb0, rows0, sem_g0).wait()

            writeback(ci1, outb, sem_w1).wait()
            return carry

        lax.fori_loop(0, n_half, pipe_body, 0)

    return sc_kernel


def kernel(input_ids, token_type_ids, word_embeddings, position_embeddings,
           token_type_embeddings, gamma, beta):
    B, S = input_ids.shape
    V, H = word_embeddings.shape
    N = B * S
    info = plsc.get_sparse_core_info()
    n_workers = info.num_cores * info.num_subcores
    C = S
    sc = _make_sc_kernel(N, S, H, V, C, n_workers)
    out = sc(
        input_ids.reshape(N),
        token_type_ids.reshape(N),
        word_embeddings,
        position_embeddings,
        token_type_embeddings,
        gamma,
        beta,
    )
    return out.reshape(B, S, H)
